# baseline (speedup 1.0000x reference)
import functools

import jax
import jax.numpy as jnp
from jax import lax
from jax.experimental import pallas as pl
from jax.experimental.pallas import tpu as pltpu
from jax.experimental.pallas import tpu_sc as plsc

B = 16384
D = 64
V = 1000000
NC = 2
NS = 16
L = 16
NW = NC * NS
BPW = B // NW
CV = 768
VFULL = 999936
NCH = VFULL // CV
CPW = 42
BH = B // 2
NTAIL = V - VFULL
SLOTS = 16
CAP = B + L

_mesh = plsc.VectorSubcoreMesh(
    core_axis_name="c", subcore_axis_name="s", num_cores=NC, num_subcores=NS
)

_params = pltpu.CompilerParams(
    use_tc_tiling_on_sc=True, needs_layout_passes=False
)


@functools.partial(
    pl.kernel,
    out_type=(
        jax.ShapeDtypeStruct((B * D,), jnp.float32),
        jax.ShapeDtypeStruct((B * D,), jnp.float32),
    ),
    mesh=_mesh,
    scratch_types=[
        pltpu.VMEM((BH,), jnp.int32),
        pltpu.VMEM((CAP,), jnp.int32),
        pltpu.VMEM((D, CV), jnp.float32),
        pltpu.VMEM((D, CV), jnp.float32),
        pltpu.VMEM((SLOTS * D,), jnp.float32),
        pltpu.SemaphoreType.DMA,
        pltpu.SemaphoreType.DMA,
        pltpu.SemaphoreType.DMA,
    ],
    compiler_params=_params,
)
def _gather_kernel(uidx_hbm, bidx_hbm, utab_hbm, btab_hbm,
                   utail_hbm, btail_hbm, uvec_hbm, bvec_hbm,
                   idx_v, list_v, chunk0_v, chunk1_v, stage_v,
                   semS, semC0, semC1):
    wid = lax.axis_index("s") * NC + lax.axis_index("c")
    lanes = lax.iota(jnp.int32, L)
    cbufs = [chunk0_v, chunk1_v]
    csems = [semC0, semC1]

    for side_idx_hbm, tab_hbm, tail_hbm, vec_hbm in (
        (uidx_hbm, utab_hbm, utail_hbm, uvec_hbm),
        (bidx_hbm, btab_hbm, btail_hbm, bvec_hbm),
    ):
        off0 = pl.multiple_of(jnp.minimum(wid, NCH - 1) * CV, 128)
        H = CV // 2
        pltpu.async_copy(tab_hbm.at[:, pl.ds(off0, H)],
                         cbufs[0].at[:, pl.ds(0, H)], csems[0])
        pltpu.async_copy(tab_hbm.at[:, pl.ds(off0 + H, H)],
                         cbufs[0].at[:, pl.ds(H, H)], csems[0])

        n_w = jnp.int32(0)
        for h in range(2):
            pltpu.sync_copy(side_idx_hbm.at[pl.ds(h * BH, BH)], idx_v)

            def bscan(v, off, _h=h):
                iv = idx_v[pl.ds(v * L, L)]
                c = jnp.right_shift(jnp.right_shift(iv, 8) * 43691, 17)
                m = ((c & (NW - 1)) == wid) & (iv < VFULL)
                cs = lax.cumsum(m.astype(jnp.int32))
                pos = off + cs - 1
                packed = (jnp.left_shift(jnp.right_shift(c, 5), 24)
                          | jnp.left_shift(iv - c * CV, 14)
                          | (_h * BH + v * L + lanes))
                plsc.store_scatter(list_v, [pos], packed, mask=m)
                return off + plsc.all_reduce_population_count(m)[0]

            n_w = lax.fori_loop(0, BH // L, bscan, n_w, unroll=2)

            @pl.when((wid // (NW // 2)) == h)
            def _(_h=h):
                def tail_body(vi, _):
                    vb = (wid % (NW // 2)) * BPW + vi * L
                    iv = idx_v[pl.ds(vb, L)]

                    def tcond(m):
                        return plsc.all_reduce_population_count(m)[0] > 0

                    def tbody(m):
                        e = plsc.all_reduce_ffs(m)[0]
                        pos = vb + e
                        idx_s = idx_v[pl.ds(pos, L)][0]
                        ti = idx_s - VFULL
                        pltpu.sync_copy(tail_hbm.at[pl.ds(ti * D, D)],
                                        stage_v.at[pl.ds(0, D)])
                        pltpu.sync_copy(
                            stage_v.at[pl.ds(0, D)],
                            vec_hbm.at[pl.ds((_h * BH + vb + e) * D, D)])
                        return m & (lanes != e)

                    lax.while_loop(tcond, tbody, iv >= VFULL)
                    return 0

                lax.fori_loop(0, BPW // L, tail_body, 0)

        nv = jnp.right_shift(n_w + (L - 1), 4)

        def scan_chunk(buf, j, n_ent0):
            def vbody(vi, n_ent1):
                vb = vi * L
                lvi = list_v[pl.ds(vb, L)]
                m0 = (jnp.right_shift(lvi, 24) == j) & ((vb + lanes) < n_w)

                def wcond(carry):
                    m, _ = carry
                    return plsc.all_reduce_population_count(m)[0] > 0

                def wbody(carry):
                    m, ne = carry
                    e = plsc.all_reduce_ffs(m)[0]
                    pos = vb + e
                    lv_s = list_v[pl.ds(pos, L)][0]
                    kk = lv_s & (B - 1)
                    colv = jnp.full(
                        (L,), jnp.right_shift(lv_s, 14) & 1023, jnp.int32)
                    s = ne & (SLOTS - 1)

                    @pl.when(ne >= SLOTS)
                    def _():
                        pltpu.make_async_copy(
                            stage_v.at[pl.ds(0, D)],
                            vec_hbm.at[pl.ds(0, D)], semS).wait()

                    for t in range(D // L):
                        dv = t * L + lanes
                        vt = plsc.load_gather(buf, [dv, colv])
                        stage_v[pl.ds(s * D + t * L, L)] = vt
                    pltpu.async_copy(stage_v.at[pl.ds(s * D, D)],
                                     vec_hbm.at[pl.ds(kk * D, D)], semS)
                    return m & (lanes != e), ne + 1

                m1, n_ent2 = lax.while_loop(wcond, wbody, (m0, n_ent1))
                return n_ent2

            return lax.fori_loop(0, nv, vbody, n_ent0)

        def chunk_of(j):
            c = jnp.minimum(wid + j * NW, NCH - 1)
            return c, pl.multiple_of(c * CV, 128)

        def pair_body(cc, n_ent0):
            n = n_ent0
            for p in (0, 1):
                j = cc * 2 + p

                @pl.when(j + 1 < CPW)
                def _():
                    _, offn = chunk_of(j + 1)
                    pltpu.async_copy(tab_hbm.at[:, pl.ds(offn, H)],
                                     cbufs[1 - p].at[:, pl.ds(0, H)],
                                     csems[1 - p])
                    pltpu.async_copy(tab_hbm.at[:, pl.ds(offn + H, H)],
                                     cbufs[1 - p].at[:, pl.ds(H, H)],
                                     csems[1 - p])

                pltpu.make_async_copy(tab_hbm.at[:, pl.ds(0, H)],
                                      cbufs[p].at[:, pl.ds(0, H)],
                                      csems[p]).wait()
                pltpu.make_async_copy(tab_hbm.at[:, pl.ds(0, H)],
                                      cbufs[p].at[:, pl.ds(0, H)],
                                      csems[p]).wait()
                n = scan_chunk(cbufs[p], j, n)
            return n

        n_ent = lax.fori_loop(0, CPW // 2, pair_body, jnp.int32(0))

        def drain(i, _):
            pltpu.make_async_copy(stage_v.at[pl.ds(0, D)],
                                  vec_hbm.at[pl.ds(0, D)], semS).wait()
            return 0

        lax.fori_loop(0, jnp.minimum(n_ent, SLOTS), drain, 0)


@functools.partial(
    pl.kernel,
    out_type=jax.ShapeDtypeStruct((B,), jnp.float32),
    mesh=_mesh,
    scratch_types=[
        pltpu.VMEM((BPW * D,), jnp.float32),
        pltpu.VMEM((BPW * D,), jnp.float32),
        pltpu.VMEM((BPW,), jnp.float32),
        pltpu.SemaphoreType.DMA,
        pltpu.SemaphoreType.DMA,
    ],
    compiler_params=_params,
)
def _dot_kernel(uvec_hbm, bvec_hbm, out_hbm, u_v, b_v, out_v, semU, semB):
    wid = lax.axis_index("s") * NC + lax.axis_index("c")
    base = wid * BPW
    lanes = lax.iota(jnp.int32, L)

    Q = BPW // 4
    for q in range(4):
        pltpu.async_copy(uvec_hbm.at[pl.ds((base + q * Q) * D, Q * D)],
                         u_v.at[pl.ds(q * Q * D, Q * D)], semU)
        pltpu.async_copy(bvec_hbm.at[pl.ds((base + q * Q) * D, Q * D)],
                         b_v.at[pl.ds(q * Q * D, Q * D)], semB)

    for q in range(4):
        pltpu.make_async_copy(uvec_hbm.at[pl.ds(0, Q * D)],
                              u_v.at[pl.ds(0, Q * D)], semU).wait()
        pltpu.make_async_copy(bvec_hbm.at[pl.ds(0, Q * D)],
                              b_v.at[pl.ds(0, Q * D)], semB).wait()

        def tile_body(t, _):
            rows = (q * Q + t * L + lanes) * D

            def d_body(d, acc):
                dvec = rows + d
                u = plsc.load_gather(u_v, [dvec])
                b = plsc.load_gather(b_v, [dvec])
                return acc + u * b

            acc = lax.fori_loop(0, D, d_body, jnp.zeros((L,), jnp.float32),
                                unroll=4)
            out_v[pl.ds(q * Q + t * L, L)] = acc
            return 0

        lax.fori_loop(0, Q // L, tile_body, 0)
    pltpu.sync_copy(out_v, out_hbm.at[pl.ds(base, BPW)])


def kernel(inputs, user_table, book_table):
    user_idx = inputs[:, 1]
    book_idx = inputs[:, 0]
    utail = user_table[VFULL:, :].reshape(-1)
    btail = book_table[VFULL:, :].reshape(-1)
    uvec, bvec = _gather_kernel(user_idx, book_idx,
                                user_table.T, book_table.T, utail, btail)
    out = _dot_kernel(uvec, bvec)
    return out.reshape(B, 1)


# speedup vs baseline: 1.0009x; 1.0009x over previous
import functools

import jax
import jax.numpy as jnp
from jax import lax
from jax.experimental import pallas as pl
from jax.experimental.pallas import tpu as pltpu
from jax.experimental.pallas import tpu_sc as plsc

B = 16384
D = 64
V = 1000000
NC = 2
NS = 16
L = 16
NW = NC * NS
BPW = B // NW
CV = 768
VFULL = 999936
NCH = VFULL // CV
CPW = 42
BH = B // 2
NTAIL = V - VFULL
SLOTS = 32
CAP = B + L

_mesh = plsc.VectorSubcoreMesh(
    core_axis_name="c", subcore_axis_name="s", num_cores=NC, num_subcores=NS
)

_params = pltpu.CompilerParams(
    use_tc_tiling_on_sc=True, needs_layout_passes=False
)


@functools.partial(
    pl.kernel,
    out_type=(
        jax.ShapeDtypeStruct((B * D,), jnp.float32),
        jax.ShapeDtypeStruct((B * D,), jnp.float32),
    ),
    mesh=_mesh,
    scratch_types=[
        pltpu.VMEM((BH,), jnp.int32),
        pltpu.VMEM((CAP,), jnp.int32),
        pltpu.VMEM((D, CV), jnp.float32),
        pltpu.VMEM((D, CV), jnp.float32),
        pltpu.VMEM((SLOTS * D,), jnp.float32),
        pltpu.SemaphoreType.DMA,
        pltpu.SemaphoreType.DMA,
        pltpu.SemaphoreType.DMA,
    ],
    compiler_params=_params,
)
def _gather_kernel(uidx_hbm, bidx_hbm, utab_hbm, btab_hbm,
                   utail_hbm, btail_hbm, uvec_hbm, bvec_hbm,
                   idx_v, list_v, chunk0_v, chunk1_v, stage_v,
                   semS, semC0, semC1):
    wid = lax.axis_index("s") * NC + lax.axis_index("c")
    lanes = lax.iota(jnp.int32, L)
    cbufs = [chunk0_v, chunk1_v]
    csems = [semC0, semC1]

    for side_idx_hbm, tab_hbm, tail_hbm, vec_hbm in (
        (uidx_hbm, utab_hbm, utail_hbm, uvec_hbm),
        (bidx_hbm, btab_hbm, btail_hbm, bvec_hbm),
    ):
        off0 = pl.multiple_of(jnp.minimum(wid, NCH - 1) * CV, 128)
        H = CV // 2
        pltpu.async_copy(tab_hbm.at[:, pl.ds(off0, H)],
                         cbufs[0].at[:, pl.ds(0, H)], csems[0])
        pltpu.async_copy(tab_hbm.at[:, pl.ds(off0 + H, H)],
                         cbufs[0].at[:, pl.ds(H, H)], csems[0])

        n_w = jnp.int32(0)
        for h in range(2):
            pltpu.sync_copy(side_idx_hbm.at[pl.ds(h * BH, BH)], idx_v)

            def bscan(v, off, _h=h):
                iv = idx_v[pl.ds(v * L, L)]
                c = jnp.right_shift(jnp.right_shift(iv, 8) * 43691, 17)
                m = ((c & (NW - 1)) == wid) & (iv < VFULL)
                cs = lax.cumsum(m.astype(jnp.int32))
                pos = off + cs - 1
                packed = (jnp.left_shift(jnp.right_shift(c, 5), 24)
                          | jnp.left_shift(iv - c * CV, 14)
                          | (_h * BH + v * L + lanes))
                plsc.store_scatter(list_v, [pos], packed, mask=m)
                return off + plsc.all_reduce_population_count(m)[0]

            n_w = lax.fori_loop(0, BH // L, bscan, n_w, unroll=2)

            @pl.when((wid // (NW // 2)) == h)
            def _(_h=h):
                def tail_body(vi, _):
                    vb = (wid % (NW // 2)) * BPW + vi * L
                    iv = idx_v[pl.ds(vb, L)]

                    def tcond(m):
                        return plsc.all_reduce_population_count(m)[0] > 0

                    def tbody(m):
                        e = plsc.all_reduce_ffs(m)[0]
                        pos = vb + e
                        idx_s = idx_v[pl.ds(pos, L)][0]
                        ti = idx_s - VFULL
                        pltpu.sync_copy(tail_hbm.at[pl.ds(ti * D, D)],
                                        stage_v.at[pl.ds(0, D)])
                        pltpu.sync_copy(
                            stage_v.at[pl.ds(0, D)],
                            vec_hbm.at[pl.ds((_h * BH + vb + e) * D, D)])
                        return m & (lanes != e)

                    lax.while_loop(tcond, tbody, iv >= VFULL)
                    return 0

                lax.fori_loop(0, BPW // L, tail_body, 0)

        nv = jnp.right_shift(n_w + (L - 1), 4)

        def scan_chunk(buf, j, n_ent0):
            def vbody(vi, n_ent1):
                vb = vi * L
                lvi = list_v[pl.ds(vb, L)]
                m0 = (jnp.right_shift(lvi, 24) == j) & ((vb + lanes) < n_w)

                def wcond(carry):
                    m, _ = carry
                    return plsc.all_reduce_population_count(m)[0] > 0

                def wbody(carry):
                    m, ne = carry
                    e = plsc.all_reduce_ffs(m)[0]
                    pos = vb + e
                    lv_s = list_v[pl.ds(pos, L)][0]
                    kk = lv_s & (B - 1)
                    colv = jnp.full(
                        (L,), jnp.right_shift(lv_s, 14) & 1023, jnp.int32)
                    s = ne & (SLOTS - 1)

                    @pl.when(ne >= SLOTS)
                    def _():
                        pltpu.make_async_copy(
                            stage_v.at[pl.ds(0, D)],
                            vec_hbm.at[pl.ds(0, D)], semS).wait()

                    for t in range(D // L):
                        dv = t * L + lanes
                        vt = plsc.load_gather(buf, [dv, colv])
                        stage_v[pl.ds(s * D + t * L, L)] = vt
                    pltpu.async_copy(stage_v.at[pl.ds(s * D, D)],
                                     vec_hbm.at[pl.ds(kk * D, D)], semS)
                    return m & (lanes != e), ne + 1

                m1, n_ent2 = lax.while_loop(wcond, wbody, (m0, n_ent1))
                return n_ent2

            return lax.fori_loop(0, nv, vbody, n_ent0)

        def chunk_of(j):
            c = jnp.minimum(wid + j * NW, NCH - 1)
            return c, pl.multiple_of(c * CV, 128)

        def pair_body(cc, n_ent0):
            n = n_ent0
            for p in (0, 1):
                j = cc * 2 + p

                @pl.when(j + 1 < CPW)
                def _():
                    _, offn = chunk_of(j + 1)
                    pltpu.async_copy(tab_hbm.at[:, pl.ds(offn, H)],
                                     cbufs[1 - p].at[:, pl.ds(0, H)],
                                     csems[1 - p])
                    pltpu.async_copy(tab_hbm.at[:, pl.ds(offn + H, H)],
                                     cbufs[1 - p].at[:, pl.ds(H, H)],
                                     csems[1 - p])

                pltpu.make_async_copy(tab_hbm.at[:, pl.ds(0, H)],
                                      cbufs[p].at[:, pl.ds(0, H)],
                                      csems[p]).wait()
                pltpu.make_async_copy(tab_hbm.at[:, pl.ds(0, H)],
                                      cbufs[p].at[:, pl.ds(0, H)],
                                      csems[p]).wait()
                n = scan_chunk(cbufs[p], j, n)
            return n

        n_ent = lax.fori_loop(0, CPW // 2, pair_body, jnp.int32(0))

        def drain(i, _):
            pltpu.make_async_copy(stage_v.at[pl.ds(0, D)],
                                  vec_hbm.at[pl.ds(0, D)], semS).wait()
            return 0

        lax.fori_loop(0, jnp.minimum(n_ent, SLOTS), drain, 0)


@functools.partial(
    pl.kernel,
    out_type=jax.ShapeDtypeStruct((B,), jnp.float32),
    mesh=_mesh,
    scratch_types=[
        pltpu.VMEM((BPW * D,), jnp.float32),
        pltpu.VMEM((BPW * D,), jnp.float32),
        pltpu.VMEM((BPW,), jnp.float32),
        pltpu.SemaphoreType.DMA,
        pltpu.SemaphoreType.DMA,
    ],
    compiler_params=_params,
)
def _dot_kernel(uvec_hbm, bvec_hbm, out_hbm, u_v, b_v, out_v, semU, semB):
    wid = lax.axis_index("s") * NC + lax.axis_index("c")
    base = wid * BPW
    lanes = lax.iota(jnp.int32, L)

    Q = BPW // 4
    for q in range(4):
        pltpu.async_copy(uvec_hbm.at[pl.ds((base + q * Q) * D, Q * D)],
                         u_v.at[pl.ds(q * Q * D, Q * D)], semU)
        pltpu.async_copy(bvec_hbm.at[pl.ds((base + q * Q) * D, Q * D)],
                         b_v.at[pl.ds(q * Q * D, Q * D)], semB)

    for q in range(4):
        pltpu.make_async_copy(uvec_hbm.at[pl.ds(0, Q * D)],
                              u_v.at[pl.ds(0, Q * D)], semU).wait()
        pltpu.make_async_copy(bvec_hbm.at[pl.ds(0, Q * D)],
                              b_v.at[pl.ds(0, Q * D)], semB).wait()

        def tile_body(t, _):
            rows = (q * Q + t * L + lanes) * D

            def d_body(d, acc):
                dvec = rows + d
                u = plsc.load_gather(u_v, [dvec])
                b = plsc.load_gather(b_v, [dvec])
                return acc + u * b

            acc = lax.fori_loop(0, D, d_body, jnp.zeros((L,), jnp.float32),
                                unroll=4)
            out_v[pl.ds(q * Q + t * L, L)] = acc
            return 0

        lax.fori_loop(0, Q // L, tile_body, 0)
    pltpu.sync_copy(out_v, out_hbm.at[pl.ds(base, BPW)])


def kernel(inputs, user_table, book_table):
    user_idx = inputs[:, 1]
    book_idx = inputs[:, 0]
    utail = user_table[VFULL:, :].reshape(-1)
    btail = book_table[VFULL:, :].reshape(-1)
    uvec, bvec = _gather_kernel(user_idx, book_idx,
                                user_table.T, book_table.T, utail, btail)
    out = _dot_kernel(uvec, bvec)
    return out.reshape(B, 1)
